# baseline (speedup 1.0000x reference)
import functools

import jax
import jax.numpy as jnp
from jax import lax
from jax.experimental import pallas as pl
from jax.experimental.pallas import tpu as pltpu
from jax.experimental.pallas import tpu_sc as plsc

_K = 64
_NROWS = 128
_N = 32768
_LANES = 16
_NWORKERS = 32
_ROWS_PER_W = _NROWS // _NWORKERS
_CHUNKS = _N // _LANES


def _f32_key(v):
    b = plsc.bitcast(v, jnp.int32)
    return b ^ (jnp.right_shift(b, 31) & jnp.int32(0x7FFFFFFF))


def _sc_body(x_hbm, outv_hbm, outi_hbm, row_a, row_b, ci_v, h1_v, h2_v,
             suf_v, wk_v, wi_v, sem_a, sem_b):
    wid = lax.axis_index("s") * 2 + lax.axis_index("c")
    lanes = lax.iota(jnp.int32, _LANES)
    ones = jnp.ones((_LANES,), jnp.int32)
    zvec = jnp.zeros((_LANES,), jnp.int32)
    l1off = lanes * 256
    l2off = lanes * 17
    base_row = wid * _ROWS_PER_W

    bufs = (row_a, row_b)
    sems = (sem_a, sem_b)
    pending = pltpu.async_copy(x_hbm.at[base_row], row_a, sem_a)

    def cl1_body(i, _):
        h1_v[pl.ds(i * _LANES, _LANES)] = zvec
        return jnp.int32(0)

    lax.fori_loop(0, 256, cl1_body, jnp.int32(0))

    def cl2_body(i, _):
        h2_v[pl.ds(i * _LANES, _LANES)] = zvec
        return jnp.int32(0)

    lax.fori_loop(0, 17, cl2_body, jnp.int32(0))

    for j in range(_ROWS_PER_W):
        cur = bufs[j % 2]
        if j + 1 < _ROWS_PER_W:
            nxt = pltpu.async_copy(
                x_hbm.at[base_row + j + 1], bufs[(j + 1) % 2],
                sems[(j + 1) % 2])
        pending.wait()


        def h1_body(c0, _):
            base = c0 * (_LANES * 16)
            vs = [cur[pl.ds(base + u * _LANES, _LANES)] for u in range(16)]
            us = [plsc.bitcast(v, jnp.uint32) for v in vs]
            dg = [plsc.bitcast(jnp.right_shift(u, 24), jnp.int32) for u in us]
            ad = [l1off + d for d in dg]
            for a in ad:
                plsc.addupdate_scatter(h1_v, [a], ones)
            return jnp.int32(0)

        lax.fori_loop(0, _CHUNKS // 16, h1_body, jnp.int32(0))

        def tot_body(l, accs):
            loaded = [h1_v[pl.ds(l * 256 + g * _LANES, _LANES)]
                      for g in range(16)]
            for g in range(16):
                h1_v[pl.ds(l * 256 + g * _LANES, _LANES)] = zvec
            return tuple(accs[g] + loaded[g] for g in range(16))

        accs = lax.fori_loop(0, _LANES, tot_body, (zvec,) * 16)
        for g in range(8):
            suf_v[pl.ds((g + 8) * _LANES, _LANES)] = accs[g]
        for g in range(8):
            suf_v[pl.ds(g * _LANES, _LANES)] = lax.rev(accs[15 - g], (0,))

        def suf_body(g2, carry):
            g = 15 - g2
            v = suf_v[pl.ds(g * _LANES, _LANES)]
            s = lax.rev(jnp.cumsum(lax.rev(v, (0,))), (0,)) + carry
            suf_v[pl.ds(g * _LANES, _LANES)] = s
            return (carry + jnp.sum(v)).astype(jnp.int32)

        lax.fori_loop(0, 16, suf_body, jnp.int32(0))

        def find_body(g, b):
            bids = g * _LANES + lanes
            sv = suf_v[pl.ds(g * _LANES, _LANES)]
            cand = jnp.where(sv >= _K, bids, -1)
            return jnp.maximum(b, jnp.max(cand)).astype(jnp.int32)

        b1 = lax.fori_loop(0, 16, find_body, jnp.int32(-1))
        thresh = jnp.left_shift(b1 - 128, 24).astype(jnp.int32)

        lo = jnp.maximum(thresh, 0).astype(jnp.int32)
        hi = jnp.where(thresh >= 0, jnp.int32(-2147483648),
                       (thresh ^ jnp.int32(0x7FFFFFFF)) + 1).astype(jnp.int32)

        def p2_body(c0, cnm1):
            base = c0 * (_LANES * 8)
            offs = [base + u * _LANES for u in range(8)]
            vs = [cur[pl.ds(o, _LANES)] for o in offs]
            bs = [plsc.bitcast(v, jnp.int32) for v in vs]
            sg = [jnp.right_shift(b, 31) for b in bs]
            ks = [b ^ (s & jnp.int32(0x7FFFFFFF)) for b, s in zip(bs, sg)]
            kp = [k >= thresh for k in ks]
            cs = [plsc.cumsum(ones, mask=k) for k in kp]
            pc = [plsc.all_reduce_population_count(k) for k in kp]
            cns = [cnm1]
            for u in range(8):
                cns.append(cns[-1] + pc[u])
            pos = [cns[u] + cs[u] for u in range(8)]
            for u in range(8):
                plsc.store_scatter(ci_v, [pos[u]], offs[u] + lanes,
                                   mask=kp[u])
            return cns[8]

        cn = lax.fori_loop(0, _CHUNKS // 8, p2_body, zvec - 1)
        m = (jnp.max(cn) + 1).astype(jnp.int32)
        c_gt = jnp.int32(0)

        def level_body(lvl, carry):
            thresh, m, _ = carry
            s = (20 - 4 * lvl).astype(jnp.int32)
            nch8 = (m + _LANES * 8 - 1) // (_LANES * 8)
            width_u = plsc.bitcast(
                jnp.full((_LANES,), jnp.left_shift(jnp.int32(1), s + 4),
                         jnp.int32), jnp.uint32)
            s_u = plsc.bitcast(jnp.full((_LANES,), s, jnp.int32), jnp.uint32)

            def hb_body(c0, _):
                base = c0 * (_LANES * 8)
                offs = [base + u * _LANES for u in range(8)]
                idxs = [ci_v[pl.ds(o, _LANES)] for o in offs]
                vls = [(o + lanes) < m for o in offs]
                kvs = [plsc.load_gather(cur, [i], mask=v)
                       for i, v in zip(idxs, vls)]
                kys = [_f32_key(kv) for kv in kvs]
                ds_ = [plsc.bitcast(k - thresh, jnp.uint32) for k in kys]
                bnd = [v & (d < width_u) for v, d in zip(vls, ds_)]
                dgs = [jnp.where(b, plsc.bitcast(jnp.right_shift(d, s_u),
                                                 jnp.int32), 16)
                       for b, d in zip(bnd, ds_)]
                for d in dgs:
                    plsc.addupdate_scatter(h2_v, [l2off + d], ones)
                return jnp.int32(0)

            lax.fori_loop(0, nch8, hb_body, jnp.int32(0))

            loaded = [h2_v[pl.ds(l * 17, _LANES)] for l in range(_LANES)]
            for l in range(_LANES):
                h2_v[pl.ds(l * 17, _LANES)] = zvec
            tot = loaded[0]
            for l in range(1, _LANES):
                tot = tot + loaded[l]
            suffix = lax.rev(jnp.cumsum(lax.rev(tot, (0,))), (0,))
            nB = jnp.sum(tot).astype(jnp.int32)
            r_l = jnp.int32(_K) - (m - nB)
            b = jnp.max(jnp.where(suffix >= r_l, lanes, -1)).astype(jnp.int32)
            sufb1 = jnp.sum(jnp.where(lanes == b + 1, suffix, 0)).astype(
                jnp.int32)
            c_gt = (m - nB) + sufb1
            thresh = (thresh + jnp.left_shift(b, s)).astype(jnp.int32)

            def cb_body(c0, cnm1):
                base = c0 * (_LANES * 8)
                offs = [base + u * _LANES for u in range(8)]
                idxs = [ci_v[pl.ds(o, _LANES)] for o in offs]
                vls = [(o + lanes) < m for o in offs]
                kvs = [plsc.load_gather(cur, [i], mask=v)
                       for i, v in zip(idxs, vls)]
                kys = [_f32_key(kv) for kv in kvs]
                kp = [v & (k >= thresh) for v, k in zip(vls, kys)]
                cs = [plsc.cumsum(ones, mask=k) for k in kp]
                pc = [plsc.all_reduce_population_count(k) for k in kp]
                cns = [cnm1]
                for u in range(8):
                    cns.append(cns[-1] + pc[u])
                pos = [cns[u] + cs[u] for u in range(8)]
                for u in range(8):
                    plsc.store_scatter(ci_v, [pos[u]], idxs[u], mask=kp[u])
                return cns[8]

            cn2 = lax.fori_loop(0, nch8, cb_body, zvec - 1)
            return (thresh, (jnp.max(cn2) + 1).astype(jnp.int32), c_gt)

        thresh, m, c_gt = lax.fori_loop(
            0, 6, level_body, (thresh, m, c_gt))

        r_fin = jnp.int32(_K) - c_gt
        nchf = (m + _LANES - 1) // _LANES

        def f_body(c, carry):
            neq, nw = carry
            idx = ci_v[pl.ds(c * _LANES, _LANES)]
            valid = (c * _LANES + lanes) < m
            kv = plsc.load_gather(cur, [idx], mask=valid)
            key = _f32_key(kv)
            gt = valid & (key > thresh)
            eq = valid & (key == thresh)
            eqi = eq.astype(jnp.int32)
            tier = neq + jnp.cumsum(eqi) - eqi
            keep = gt | (eq & (tier < r_fin))
            ki = keep.astype(jnp.int32)
            pos = nw + jnp.cumsum(ki) - ki + (j * _K)
            plsc.store_scatter(wi_v, [pos], idx, mask=keep)
            return (neq + plsc.all_reduce_population_count(eq),
                    nw + plsc.all_reduce_population_count(keep))

        lax.fori_loop(0, nchf, f_body, (zvec, zvec))

        for c in range(_K // _LANES):
            iv = wi_v[pl.ds(j * _K + c * _LANES, _LANES)]
            wk_v[pl.ds(j * _K + c * _LANES, _LANES)] = plsc.load_gather(
                cur, [iv])

        if j + 1 < _ROWS_PER_W:
            pending = nxt

    pltpu.sync_copy(wk_v, outv_hbm.at[pl.ds(wid * (_ROWS_PER_W * _K),
                                            _ROWS_PER_W * _K)])
    pltpu.sync_copy(wi_v, outi_hbm.at[pl.ds(wid * (_ROWS_PER_W * _K),
                                            _ROWS_PER_W * _K)])


_sc_topk = functools.partial(
    pl.kernel,
    out_type=(
        jax.ShapeDtypeStruct((_NROWS * _K,), jnp.float32),
        jax.ShapeDtypeStruct((_NROWS * _K,), jnp.int32),
    ),
    mesh=plsc.VectorSubcoreMesh(core_axis_name="c", subcore_axis_name="s"),
    compiler_params=pltpu.CompilerParams(
        needs_layout_passes=False, use_tc_tiling_on_sc=True),
    scratch_types=[
        pltpu.VMEM((_N,), jnp.float32),
        pltpu.VMEM((_N,), jnp.float32),
        pltpu.VMEM((_N,), jnp.int32),
        pltpu.VMEM((16 * 256,), jnp.int32),
        pltpu.VMEM((16 * 17,), jnp.int32),
        pltpu.VMEM((256,), jnp.int32),
        pltpu.VMEM((_ROWS_PER_W * _K,), jnp.float32),
        pltpu.VMEM((_ROWS_PER_W * _K,), jnp.int32),
        pltpu.SemaphoreType.DMA,
        pltpu.SemaphoreType.DMA,
    ],
)(_sc_body)


def _tc_sort_body(k_ref, i_ref, vo_ref, io_ref):
    k = k_ref[...]
    ix = i_ref[...]
    lane = lax.broadcasted_iota(jnp.int32, (_NROWS, _K), 1)
    for kk in [2, 4, 8, 16, 32, 64]:
        j = kk // 2
        while j >= 1:
            lk = jnp.concatenate([k[:, j:], k[:, :j]], axis=1)
            rk = jnp.concatenate([k[:, _K - j:], k[:, :_K - j]], axis=1)
            li = jnp.concatenate([ix[:, j:], ix[:, :j]], axis=1)
            ri = jnp.concatenate([ix[:, _K - j:], ix[:, :_K - j]], axis=1)
            bitclear = (lane & j) == 0
            pk = jnp.where(bitclear, lk, rk)
            pi = jnp.where(bitclear, li, ri)
            first = (k > pk) | ((k == pk) & (ix < pi))
            forward = (lane & kk) == 0
            take = bitclear ^ first ^ (~forward)
            k = jnp.where(take, pk, k)
            ix = jnp.where(take, pi, ix)
            j //= 2
    vo_ref[...] = k
    io_ref[...] = ix


_tc_sort = pl.pallas_call(
    _tc_sort_body,
    out_shape=(
        jax.ShapeDtypeStruct((_NROWS, _K), jnp.float32),
        jax.ShapeDtypeStruct((_NROWS, _K), jnp.int32),
    ),
)


def kernel(x):
    wv, wi = _sc_topk(x)
    return _tc_sort(wv.reshape(_NROWS, _K), wi.reshape(_NROWS, _K))


# speedup vs baseline: 1.1492x; 1.1492x over previous
import functools

import jax
import jax.numpy as jnp
from jax import lax
from jax.experimental import pallas as pl
from jax.experimental.pallas import tpu as pltpu
from jax.experimental.pallas import tpu_sc as plsc

_K = 64
_NROWS = 128
_N = 32768
_LANES = 16
_NWORKERS = 32
_ROWS_PER_W = _NROWS // _NWORKERS
_CHUNKS = _N // _LANES


def _f32_key(v):
    b = plsc.bitcast(v, jnp.int32)
    return b ^ (jnp.right_shift(b, 31) & jnp.int32(0x7FFFFFFF))


def _sc_body(x_hbm, outv_hbm, outi_hbm, row_a, row_b, ci_v, h1_v, h2_v,
             suf_v, wk_v, wi_v, sem_a, sem_b):
    wid = lax.axis_index("s") * 2 + lax.axis_index("c")
    lanes = lax.iota(jnp.int32, _LANES)
    ones = jnp.ones((_LANES,), jnp.int32)
    zvec = jnp.zeros((_LANES,), jnp.int32)
    l1off = lanes * 256
    l2off = lanes * 17
    base_row = wid * _ROWS_PER_W

    bufs = (row_a, row_b)
    sems = (sem_a, sem_b)
    pending = pltpu.async_copy(x_hbm.at[base_row], row_a, sem_a)

    def cl1_body(i, _):
        h1_v[pl.ds(i * _LANES, _LANES)] = zvec
        return jnp.int32(0)

    lax.fori_loop(0, 256, cl1_body, jnp.int32(0))

    def cl2_body(i, _):
        h2_v[pl.ds(i * _LANES, _LANES)] = zvec
        return jnp.int32(0)

    lax.fori_loop(0, 17, cl2_body, jnp.int32(0))

    for j in range(_ROWS_PER_W):
        cur = bufs[j % 2]
        if j + 1 < _ROWS_PER_W:
            nxt = pltpu.async_copy(
                x_hbm.at[base_row + j + 1], bufs[(j + 1) % 2],
                sems[(j + 1) % 2])
        pending.wait()


        def h1_body(c0, _):
            base = c0 * (_LANES * 16)
            vs = [cur[pl.ds(base + u * _LANES, _LANES)] for u in range(16)]
            bs = [plsc.bitcast(v, jnp.int32) for v in vs]
            sg = [jnp.right_shift(b, 31) for b in bs]
            fl = [s | jnp.int32(-2147483648) for s in sg]
            us = [plsc.bitcast(b ^ f, jnp.uint32) for b, f in zip(bs, fl)]
            dg = [plsc.bitcast(jnp.right_shift(u, 24), jnp.int32) for u in us]
            ad = [l1off + d for d in dg]
            for a in ad:
                plsc.addupdate_scatter(h1_v, [a], ones)
            return jnp.int32(0)

        lax.fori_loop(0, _CHUNKS // 16, h1_body, jnp.int32(0))

        def tot_body(l, accs):
            loaded = [h1_v[pl.ds(l * 256 + g * _LANES, _LANES)]
                      for g in range(16)]
            for g in range(16):
                h1_v[pl.ds(l * 256 + g * _LANES, _LANES)] = zvec
            return tuple(accs[g] + loaded[g] for g in range(16))

        accs = lax.fori_loop(0, _LANES, tot_body, (zvec,) * 16)
        for g in range(16):
            suf_v[pl.ds(g * _LANES, _LANES)] = accs[g]

        def suf_body(g2, carry):
            g = 15 - g2
            v = suf_v[pl.ds(g * _LANES, _LANES)]
            s = lax.rev(jnp.cumsum(lax.rev(v, (0,))), (0,)) + carry
            suf_v[pl.ds(g * _LANES, _LANES)] = s
            return (carry + jnp.sum(v)).astype(jnp.int32)

        lax.fori_loop(0, 16, suf_body, jnp.int32(0))

        def find_body(g, b):
            bids = g * _LANES + lanes
            sv = suf_v[pl.ds(g * _LANES, _LANES)]
            cand = jnp.where(sv >= _K, bids, -1)
            return jnp.maximum(b, jnp.max(cand)).astype(jnp.int32)

        b1 = lax.fori_loop(0, 16, find_body, jnp.int32(-1))
        thresh = jnp.left_shift(b1 - 128, 24).astype(jnp.int32)

        lo = jnp.maximum(thresh, 0).astype(jnp.int32)
        hi = jnp.where(thresh >= 0, jnp.int32(-2147483648),
                       (thresh ^ jnp.int32(0x7FFFFFFF)) + 1).astype(jnp.int32)

        def p2_body(c0, cnm1):
            base = c0 * (_LANES * 8)
            offs = [base + u * _LANES for u in range(8)]
            vs = [cur[pl.ds(o, _LANES)] for o in offs]
            bs = [plsc.bitcast(v, jnp.int32) for v in vs]
            kp = [(b >= lo) | (b < hi) for b in bs]
            cs = [plsc.cumsum(ones, mask=k) for k in kp]
            pc = [plsc.all_reduce_population_count(k) for k in kp]
            cns = [cnm1]
            for u in range(8):
                cns.append(cns[-1] + pc[u])
            pos = [cns[u] + cs[u] for u in range(8)]
            for u in range(8):
                plsc.store_scatter(ci_v, [pos[u]], offs[u] + lanes,
                                   mask=kp[u])
            return cns[8]

        cn = lax.fori_loop(0, _CHUNKS // 8, p2_body, zvec - 1)
        m = (jnp.max(cn) + 1).astype(jnp.int32)
        c_gt = jnp.int32(0)

        def level_body(lvl, carry):
            thresh, m, _ = carry
            s = (20 - 4 * lvl).astype(jnp.int32)
            nch8 = (m + _LANES * 8 - 1) // (_LANES * 8)
            width_u = plsc.bitcast(
                jnp.full((_LANES,), jnp.left_shift(jnp.int32(1), s + 4),
                         jnp.int32), jnp.uint32)
            s_u = plsc.bitcast(jnp.full((_LANES,), s, jnp.int32), jnp.uint32)

            def hb_body(c0, _):
                base = c0 * (_LANES * 8)
                offs = [base + u * _LANES for u in range(8)]
                idxs = [ci_v[pl.ds(o, _LANES)] for o in offs]
                vls = [(o + lanes) < m for o in offs]
                kvs = [plsc.load_gather(cur, [i], mask=v)
                       for i, v in zip(idxs, vls)]
                kys = [_f32_key(kv) for kv in kvs]
                ds_ = [plsc.bitcast(k - thresh, jnp.uint32) for k in kys]
                bnd = [v & (d < width_u) for v, d in zip(vls, ds_)]
                dgs = [jnp.where(b, plsc.bitcast(jnp.right_shift(d, s_u),
                                                 jnp.int32), 16)
                       for b, d in zip(bnd, ds_)]
                for d in dgs:
                    plsc.addupdate_scatter(h2_v, [l2off + d], ones)
                return jnp.int32(0)

            lax.fori_loop(0, nch8, hb_body, jnp.int32(0))

            loaded = [h2_v[pl.ds(l * 17, _LANES)] for l in range(_LANES)]
            for l in range(_LANES):
                h2_v[pl.ds(l * 17, _LANES)] = zvec
            tot = loaded[0]
            for l in range(1, _LANES):
                tot = tot + loaded[l]
            suffix = lax.rev(jnp.cumsum(lax.rev(tot, (0,))), (0,))
            nB = jnp.sum(tot).astype(jnp.int32)
            r_l = jnp.int32(_K) - (m - nB)
            b = jnp.max(jnp.where(suffix >= r_l, lanes, -1)).astype(jnp.int32)
            sufb1 = jnp.sum(jnp.where(lanes == b + 1, suffix, 0)).astype(
                jnp.int32)
            c_gt = (m - nB) + sufb1
            thresh = (thresh + jnp.left_shift(b, s)).astype(jnp.int32)

            def cb_body(c0, cnm1):
                base = c0 * (_LANES * 8)
                offs = [base + u * _LANES for u in range(8)]
                idxs = [ci_v[pl.ds(o, _LANES)] for o in offs]
                vls = [(o + lanes) < m for o in offs]
                kvs = [plsc.load_gather(cur, [i], mask=v)
                       for i, v in zip(idxs, vls)]
                kys = [_f32_key(kv) for kv in kvs]
                kp = [v & (k >= thresh) for v, k in zip(vls, kys)]
                cs = [plsc.cumsum(ones, mask=k) for k in kp]
                pc = [plsc.all_reduce_population_count(k) for k in kp]
                cns = [cnm1]
                for u in range(8):
                    cns.append(cns[-1] + pc[u])
                pos = [cns[u] + cs[u] for u in range(8)]
                for u in range(8):
                    plsc.store_scatter(ci_v, [pos[u]], idxs[u], mask=kp[u])
                return cns[8]

            cn2 = lax.fori_loop(0, nch8, cb_body, zvec - 1)
            return (thresh, (jnp.max(cn2) + 1).astype(jnp.int32), c_gt)

        thresh, m, c_gt = lax.fori_loop(
            0, 6, level_body, (thresh, m, c_gt))

        r_fin = jnp.int32(_K) - c_gt
        nchf = (m + _LANES - 1) // _LANES

        def f_body(c, carry):
            neq, nw = carry
            idx = ci_v[pl.ds(c * _LANES, _LANES)]
            valid = (c * _LANES + lanes) < m
            kv = plsc.load_gather(cur, [idx], mask=valid)
            key = _f32_key(kv)
            gt = valid & (key > thresh)
            eq = valid & (key == thresh)
            eqi = eq.astype(jnp.int32)
            tier = neq + jnp.cumsum(eqi) - eqi
            keep = gt | (eq & (tier < r_fin))
            ki = keep.astype(jnp.int32)
            pos = nw + jnp.cumsum(ki) - ki + (j * _K)
            plsc.store_scatter(wi_v, [pos], idx, mask=keep)
            return (neq + plsc.all_reduce_population_count(eq),
                    nw + plsc.all_reduce_population_count(keep))

        lax.fori_loop(0, nchf, f_body, (zvec, zvec))

        for c in range(_K // _LANES):
            iv = wi_v[pl.ds(j * _K + c * _LANES, _LANES)]
            wk_v[pl.ds(j * _K + c * _LANES, _LANES)] = plsc.load_gather(
                cur, [iv])

        if j + 1 < _ROWS_PER_W:
            pending = nxt

    pltpu.sync_copy(wk_v, outv_hbm.at[pl.ds(wid * (_ROWS_PER_W * _K),
                                            _ROWS_PER_W * _K)])
    pltpu.sync_copy(wi_v, outi_hbm.at[pl.ds(wid * (_ROWS_PER_W * _K),
                                            _ROWS_PER_W * _K)])


_sc_topk = functools.partial(
    pl.kernel,
    out_type=(
        jax.ShapeDtypeStruct((_NROWS * _K,), jnp.float32),
        jax.ShapeDtypeStruct((_NROWS * _K,), jnp.int32),
    ),
    mesh=plsc.VectorSubcoreMesh(core_axis_name="c", subcore_axis_name="s"),
    compiler_params=pltpu.CompilerParams(
        needs_layout_passes=False, use_tc_tiling_on_sc=True),
    scratch_types=[
        pltpu.VMEM((_N,), jnp.float32),
        pltpu.VMEM((_N,), jnp.float32),
        pltpu.VMEM((_N,), jnp.int32),
        pltpu.VMEM((16 * 256,), jnp.int32),
        pltpu.VMEM((16 * 17,), jnp.int32),
        pltpu.VMEM((256,), jnp.int32),
        pltpu.VMEM((_ROWS_PER_W * _K,), jnp.float32),
        pltpu.VMEM((_ROWS_PER_W * _K,), jnp.int32),
        pltpu.SemaphoreType.DMA,
        pltpu.SemaphoreType.DMA,
    ],
)(_sc_body)


def _tc_sort_body(k_ref, i_ref, vo_ref, io_ref):
    k = k_ref[...]
    ix = i_ref[...]
    lane = lax.broadcasted_iota(jnp.int32, (_NROWS, _K), 1)
    for kk in [2, 4, 8, 16, 32, 64]:
        j = kk // 2
        while j >= 1:
            lk = jnp.concatenate([k[:, j:], k[:, :j]], axis=1)
            rk = jnp.concatenate([k[:, _K - j:], k[:, :_K - j]], axis=1)
            li = jnp.concatenate([ix[:, j:], ix[:, :j]], axis=1)
            ri = jnp.concatenate([ix[:, _K - j:], ix[:, :_K - j]], axis=1)
            bitclear = (lane & j) == 0
            pk = jnp.where(bitclear, lk, rk)
            pi = jnp.where(bitclear, li, ri)
            first = (k > pk) | ((k == pk) & (ix < pi))
            forward = (lane & kk) == 0
            take = bitclear ^ first ^ (~forward)
            k = jnp.where(take, pk, k)
            ix = jnp.where(take, pi, ix)
            j //= 2
    vo_ref[...] = k
    io_ref[...] = ix


_tc_sort = pl.pallas_call(
    _tc_sort_body,
    out_shape=(
        jax.ShapeDtypeStruct((_NROWS, _K), jnp.float32),
        jax.ShapeDtypeStruct((_NROWS, _K), jnp.int32),
    ),
)


def kernel(x):
    wv, wi = _sc_topk(x)
    return _tc_sort(wv.reshape(_NROWS, _K), wi.reshape(_NROWS, _K))
